# Initial kernel scaffold; baseline (speedup 1.0000x reference)
import jax
import jax.numpy as jnp
from jax.experimental import pallas as pl


def kernel(x, rule_indices, rule_weights):
    raise NotImplementedError("write your pallas kernel here")


# speedup vs baseline: 2.2507x; 2.2507x over previous
import functools

import jax
import jax.numpy as jnp
from jax import lax
from jax.experimental import pallas as pl
from jax.experimental.pallas import tpu as pltpu
from jax.experimental.pallas import tpu_sc as plsc

N_STATES = 4096
N_RULES = 1638400

NUM_SC = 2
NUM_TILES = 16
LANES = 16

W_CHUNK = 1048576
N_CHUNKS = (N_STATES * N_STATES) // W_CHUNK
CHUNKS_PER_SC = N_CHUNKS // NUM_SC
W_TILE = W_CHUNK // NUM_TILES

E_TILE = N_RULES // NUM_TILES
BE = 2048
N_BATCH = E_TILE // BE
N_GROUPS = BE // 128
VPG = 128 // LANES

ZBUF = 4096


def _scatter_body(body_hbm, head_hbm, w_hbm, adj_out, acc, bvec, hvec, wvec,
                  idx128, w128, zbuf, stage):
    c = lax.axis_index("c")
    s = lax.axis_index("s")
    iota = lax.iota(jnp.int32, LANES)

    def _z(i, carry):
        zbuf[pl.ds(i * LANES, LANES)] = jnp.zeros((LANES,), jnp.float32)
        return carry
    lax.fori_loop(0, ZBUF // LANES, _z, 0)

    def chunk_body(k, carry):
        chunk = k * NUM_SC + c
        base = chunk * W_CHUNK

        def _zero(j, carry2):
            pltpu.sync_copy(zbuf, acc.at[pl.ds(s * W_TILE + j * ZBUF, ZBUF)])
            return carry2
        lax.fori_loop(0, W_TILE // ZBUF, _zero, 0)
        plsc.subcore_barrier()

        def batch_body(j, carry2):
            ebase = s * E_TILE + j * BE
            pltpu.sync_copy(body_hbm.at[pl.ds(ebase, BE)], bvec)
            pltpu.sync_copy(head_hbm.at[pl.ds(ebase, BE)], hvec)
            pltpu.sync_copy(w_hbm.at[pl.ds(ebase, BE)], wvec)
            for g in range(N_GROUPS):
                for v in range(VPG):
                    off = g * 128 + v * LANES
                    b16 = bvec[pl.ds(off, LANES)]
                    h16 = hvec[pl.ds(off, LANES)]
                    w16 = wvec[pl.ds(off, LANES)]
                    local = (b16 << 12) + h16 - base
                    inb = plsc.bitcast(local, jnp.uint32) < jnp.uint32(W_CHUNK)
                    dummy = iota + off
                    idx128[pl.ds(v * LANES, LANES)] = jnp.where(inb, local, dummy)
                    w128[pl.ds(v * LANES, LANES)] = jnp.where(inb, w16, 0.0)
                pltpu.sync_copy(w128, acc.at[idx128], add=True)
            return carry2
        lax.fori_loop(0, N_BATCH, batch_body, 0)
        plsc.subcore_barrier()

        def _wout(j, carry2):
            off = s * W_TILE + j * ZBUF
            pltpu.sync_copy(acc.at[pl.ds(off, ZBUF)], stage)
            pltpu.sync_copy(stage, adj_out.at[pl.ds(base + off, ZBUF)])
            return carry2
        lax.fori_loop(0, W_TILE // ZBUF, _wout, 0)
        return carry
    lax.fori_loop(0, CHUNKS_PER_SC, chunk_body, 0)


_scatter_sc = functools.partial(
    pl.kernel,
    out_type=jax.ShapeDtypeStruct((N_STATES * N_STATES,), jnp.float32),
    mesh=plsc.VectorSubcoreMesh(core_axis_name="c", subcore_axis_name="s"),
    scratch_types=[
        pltpu.VMEM_SHARED((W_CHUNK,), jnp.float32),
        pltpu.VMEM((BE,), jnp.int32),
        pltpu.VMEM((BE,), jnp.int32),
        pltpu.VMEM((BE,), jnp.float32),
        pltpu.VMEM((128,), jnp.int32),
        pltpu.VMEM((128,), jnp.float32),
        pltpu.VMEM((ZBUF,), jnp.float32),
        pltpu.VMEM((ZBUF,), jnp.float32),
    ],
)(_scatter_body)


def _mm_body(x_ref, a_ref, o_ref):
    acc = jnp.dot(x_ref[...], a_ref[...],
                  preferred_element_type=jnp.float32,
                  precision=lax.Precision.HIGHEST)
    o_ref[...] = jax.nn.sigmoid(acc)


BM = 256
BN = 512


def _matmul_tc(x, adj):
    m = x.shape[0]
    return pl.pallas_call(
        _mm_body,
        grid=(N_STATES // BN, m // BM),
        in_specs=[
            pl.BlockSpec((BM, N_STATES), lambda j, i: (i, 0)),
            pl.BlockSpec((N_STATES, BN), lambda j, i: (0, j)),
        ],
        out_specs=pl.BlockSpec((BM, BN), lambda j, i: (i, j)),
        out_shape=jax.ShapeDtypeStruct((m, N_STATES), jnp.float32),
    )(x, adj)


def kernel(x, rule_indices, rule_weights):
    body = rule_indices[0]
    head = rule_indices[1]
    adj_flat = _scatter_sc(body, head, rule_weights)
    adj = adj_flat.reshape(N_STATES, N_STATES)
    return _matmul_tc(x, adj)


# speedup vs baseline: 5.5665x; 2.4732x over previous
import functools

import jax
import jax.numpy as jnp
from jax import lax
from jax.experimental import pallas as pl
from jax.experimental.pallas import tpu as pltpu
from jax.experimental.pallas import tpu_sc as plsc

N_STATES = 4096
N_RULES = 1638400
ADJ_WORDS = N_STATES * N_STATES

NUM_SC = 2
NUM_TILES = 16
LANES = 16

W_PASS = 1507328
N_PASS = -(-ADJ_WORDS // W_PASS)

E_TILE = N_RULES // NUM_TILES
BE = 2048
N_PAIR = E_TILE // (2 * BE)
VPB = BE // LANES

ZBUF = 4096


def _scatter_body(body_hbm, head_hbm, w_hbm, adj_out, acc, bvec, hvec, wvec,
                  idx2d, w2d, zbuf, stage, ld_sem, st_sem):
    c = lax.axis_index("c")
    s = lax.axis_index("s")
    iota = lax.iota(jnp.int32, LANES)

    def _z(i, carry):
        zbuf[pl.ds(i * LANES, LANES)] = jnp.zeros((LANES,), jnp.float32)
        return carry
    lax.fori_loop(0, ZBUF // LANES, _z, 0)

    def _ld(j, p):
        eb = s * E_TILE + j * BE
        pltpu.async_copy(body_hbm.at[pl.ds(eb, BE)], bvec.at[p], ld_sem.at[p])
        pltpu.async_copy(head_hbm.at[pl.ds(eb, BE)], hvec.at[p], ld_sem.at[p])
        pltpu.async_copy(w_hbm.at[pl.ds(eb, BE)], wvec.at[p], ld_sem.at[p])

    def _ld_wait(j, p):
        eb = s * E_TILE + j * BE
        pltpu.make_async_copy(body_hbm.at[pl.ds(eb, BE)], bvec.at[p], ld_sem.at[p]).wait()
        pltpu.make_async_copy(head_hbm.at[pl.ds(eb, BE)], hvec.at[p], ld_sem.at[p]).wait()
        pltpu.make_async_copy(w_hbm.at[pl.ds(eb, BE)], wvec.at[p], ld_sem.at[p]).wait()

    def pass_body(k, carry):
        base = (k * NUM_SC + c) * W_PASS
        n_words = jnp.minimum(W_PASS, ADJ_WORDS - base)
        tile_share = n_words // NUM_TILES
        n_slabs = tile_share // ZBUF

        def _zero(j, carry2):
            off = pl.multiple_of(s * tile_share + j * ZBUF, ZBUF)
            pltpu.sync_copy(zbuf, acc.at[pl.ds(off, ZBUF)])
            return carry2
        lax.fori_loop(0, n_slabs, _zero, 0)
        plsc.subcore_barrier()

        def _compute(j, p):
            for v in range(VPB):
                off = v * LANES
                b16 = bvec[p, pl.ds(off, LANES)]
                h16 = hvec[p, pl.ds(off, LANES)]
                w16 = wvec[p, pl.ds(off, LANES)]
                local = (b16 << 12) + h16 - base
                inb = plsc.bitcast(local, jnp.uint32) < jnp.uint32(W_PASS)
                dummy = iota + off
                idx2d[p, v // 8, pl.ds((v % 8) * LANES, LANES)] = (
                    jnp.where(inb, local, dummy))
                w2d[p, v // 8, pl.ds((v % 8) * LANES, LANES)] = (
                    jnp.where(inb, w16, 0.0))

        def _scat_start(p):
            for g in range(BE // 128):
                pltpu.async_copy(w2d.at[p, g], acc.at[idx2d.at[p, g]],
                                 st_sem.at[p], add=True)

        def _scat_wait(p):
            for g in range(BE // 128):
                pltpu.make_async_copy(w2d.at[p, g], acc.at[idx2d.at[p, g]],
                                      st_sem.at[p]).wait()

        _ld(0, 0)

        def pair_body(i, carry2):
            _ld_wait(2 * i, 0)
            _ld(2 * i + 1, 1)

            @pl.when(i > 0)
            def _():
                _scat_wait(0)
            _compute(2 * i, 0)
            _scat_start(0)

            _ld_wait(2 * i + 1, 1)

            @pl.when(i < N_PAIR - 1)
            def _():
                _ld(2 * i + 2, 0)

            @pl.when(i > 0)
            def _():
                _scat_wait(1)
            _compute(2 * i + 1, 1)
            _scat_start(1)
            return carry2
        lax.fori_loop(0, N_PAIR, pair_body, 0)
        _scat_wait(0)
        _scat_wait(1)
        plsc.subcore_barrier()

        def _wout(j, carry2):
            off = pl.multiple_of(s * tile_share + j * ZBUF, ZBUF)
            ooff = pl.multiple_of(base + off, ZBUF)
            pltpu.sync_copy(acc.at[pl.ds(off, ZBUF)], stage)
            pltpu.sync_copy(stage, adj_out.at[pl.ds(ooff, ZBUF)])
            return carry2
        lax.fori_loop(0, n_slabs, _wout, 0)
        plsc.subcore_barrier()
        return carry
    n_my_passes = (N_PASS + 1 - c) // 2
    lax.fori_loop(0, n_my_passes, pass_body, 0)


_scatter_sc = functools.partial(
    pl.kernel,
    out_type=jax.ShapeDtypeStruct((ADJ_WORDS,), jnp.float32),
    mesh=plsc.VectorSubcoreMesh(core_axis_name="c", subcore_axis_name="s"),
    scratch_types=[
        pltpu.VMEM_SHARED((W_PASS,), jnp.float32),
        pltpu.VMEM((2, BE), jnp.int32),
        pltpu.VMEM((2, BE), jnp.int32),
        pltpu.VMEM((2, BE), jnp.float32),
        pltpu.VMEM((2, BE // 128, 128), jnp.int32),
        pltpu.VMEM((2, BE // 128, 128), jnp.float32),
        pltpu.VMEM((ZBUF,), jnp.float32),
        pltpu.VMEM((ZBUF,), jnp.float32),
        pltpu.SemaphoreType.DMA((2,)),
        pltpu.SemaphoreType.DMA((2,)),
    ],
)(_scatter_body)


def _mm_body(x_ref, a_ref, o_ref):
    acc = jnp.dot(x_ref[...], a_ref[...],
                  preferred_element_type=jnp.float32,
                  precision=lax.Precision.HIGHEST)
    o_ref[...] = jax.nn.sigmoid(acc)


BM = 256
BN = 512


def _matmul_tc(x, adj):
    m = x.shape[0]
    return pl.pallas_call(
        _mm_body,
        grid=(N_STATES // BN, m // BM),
        in_specs=[
            pl.BlockSpec((BM, N_STATES), lambda j, i: (i, 0)),
            pl.BlockSpec((N_STATES, BN), lambda j, i: (0, j)),
        ],
        out_specs=pl.BlockSpec((BM, BN), lambda j, i: (i, j)),
        out_shape=jax.ShapeDtypeStruct((m, N_STATES), jnp.float32),
    )(x, adj)


def kernel(x, rule_indices, rule_weights):
    body = rule_indices[0]
    head = rule_indices[1]
    adj_flat = _scatter_sc(body, head, rule_weights)
    adj = adj_flat.reshape(N_STATES, N_STATES)
    return _matmul_tc(x, adj)


# speedup vs baseline: 6.0961x; 1.0951x over previous
import functools

import jax
import jax.numpy as jnp
from jax import lax
from jax.experimental import pallas as pl
from jax.experimental.pallas import tpu as pltpu
from jax.experimental.pallas import tpu_sc as plsc

N_STATES = 4096
N_RULES = 1638400
ADJ_WORDS = N_STATES * N_STATES

NUM_SC = 2
NUM_TILES = 16
LANES = 16

W_PASS = 1507328
N_PASS = -(-ADJ_WORDS // W_PASS)

E_TILE = N_RULES // NUM_TILES
BE = 2048
N_PAIR = E_TILE // (2 * BE)
VPB = BE // LANES

ZBUF = 4096


def _scatter_body(body_hbm, head_hbm, w_hbm, adj_out, acc, bvec, hvec, wvec,
                  idx2d, w2d, zbuf, stage, ld_sem, st_sem):
    c = lax.axis_index("c")
    s = lax.axis_index("s")
    iota = lax.iota(jnp.int32, LANES)

    def _z(i, carry):
        zbuf[pl.ds(i * LANES, LANES)] = jnp.zeros((LANES,), jnp.float32)
        return carry
    lax.fori_loop(0, ZBUF // LANES, _z, 0)

    def _ld(j, p):
        eb = s * E_TILE + j * BE
        pltpu.async_copy(body_hbm.at[pl.ds(eb, BE)], bvec.at[p], ld_sem.at[p])
        pltpu.async_copy(head_hbm.at[pl.ds(eb, BE)], hvec.at[p], ld_sem.at[p])
        pltpu.async_copy(w_hbm.at[pl.ds(eb, BE)], wvec.at[p], ld_sem.at[p])

    def _ld_wait(j, p):
        eb = s * E_TILE + j * BE
        pltpu.make_async_copy(body_hbm.at[pl.ds(eb, BE)], bvec.at[p], ld_sem.at[p]).wait()
        pltpu.make_async_copy(head_hbm.at[pl.ds(eb, BE)], hvec.at[p], ld_sem.at[p]).wait()
        pltpu.make_async_copy(w_hbm.at[pl.ds(eb, BE)], wvec.at[p], ld_sem.at[p]).wait()

    def pass_body(k, carry):
        base = (k * NUM_SC + c) * W_PASS
        n_words = jnp.minimum(W_PASS, ADJ_WORDS - base)
        tile_share = n_words // NUM_TILES
        n_slabs = tile_share // ZBUF

        def _zero(j, carry2):
            off = pl.multiple_of(s * tile_share + j * ZBUF, ZBUF)
            pltpu.sync_copy(zbuf, acc.at[pl.ds(off, ZBUF)])
            return carry2
        lax.fori_loop(0, n_slabs, _zero, 0)
        plsc.subcore_barrier()

        def _compute(j, p):
            for v in range(VPB):
                off = v * LANES
                b16 = bvec[p, pl.ds(off, LANES)]
                h16 = hvec[p, pl.ds(off, LANES)]
                w16 = wvec[p, pl.ds(off, LANES)]
                local = (b16 << 12) + h16 - base
                inb = plsc.bitcast(local, jnp.uint32) < jnp.uint32(W_PASS)
                dummy = iota + off
                idx2d[p, v // 8, pl.ds((v % 8) * LANES, LANES)] = (
                    jnp.where(inb, local, dummy))
                w2d[p, v // 8, pl.ds((v % 8) * LANES, LANES)] = (
                    jnp.where(inb, w16, 0.0))

        def _scat_start(p):
            for g in range(BE // 128):
                pltpu.async_copy(w2d.at[p, g], acc.at[idx2d.at[p, g]],
                                 st_sem.at[p], add=True)

        def _scat_wait(p):
            for g in range(BE // 128):
                pltpu.make_async_copy(w2d.at[p, g], acc.at[idx2d.at[p, g]],
                                      st_sem.at[p]).wait()

        _ld(0, 0)

        def pair_body(i, carry2):
            _ld_wait(2 * i, 0)
            _ld(2 * i + 1, 1)

            @pl.when(i > 0)
            def _():
                _scat_wait(0)
            _compute(2 * i, 0)
            _scat_start(0)

            _ld_wait(2 * i + 1, 1)

            @pl.when(i < N_PAIR - 1)
            def _():
                _ld(2 * i + 2, 0)

            @pl.when(i > 0)
            def _():
                _scat_wait(1)
            _compute(2 * i + 1, 1)
            _scat_start(1)
            return carry2
        lax.fori_loop(0, N_PAIR, pair_body, 0)
        _scat_wait(0)
        _scat_wait(1)
        plsc.subcore_barrier()

        def _wout(j, carry2):
            off = pl.multiple_of(s * tile_share + j * ZBUF, ZBUF)
            row = (base + off) >> 12
            pltpu.sync_copy(acc.at[pl.ds(off, ZBUF)], stage)
            pltpu.sync_copy(stage, adj_out.at[row])
            return carry2
        lax.fori_loop(0, n_slabs, _wout, 0)
        plsc.subcore_barrier()
        return carry
    n_my_passes = (N_PASS + 1 - c) // 2
    lax.fori_loop(0, n_my_passes, pass_body, 0)


_scatter_sc = functools.partial(
    pl.kernel,
    out_type=jax.ShapeDtypeStruct((N_STATES, N_STATES), jnp.float32),
    mesh=plsc.VectorSubcoreMesh(core_axis_name="c", subcore_axis_name="s"),
    scratch_types=[
        pltpu.VMEM_SHARED((W_PASS,), jnp.float32),
        pltpu.VMEM((2, BE), jnp.int32),
        pltpu.VMEM((2, BE), jnp.int32),
        pltpu.VMEM((2, BE), jnp.float32),
        pltpu.VMEM((2, BE // 128, 128), jnp.int32),
        pltpu.VMEM((2, BE // 128, 128), jnp.float32),
        pltpu.VMEM((ZBUF,), jnp.float32),
        pltpu.VMEM((ZBUF,), jnp.float32),
        pltpu.SemaphoreType.DMA((2,)),
        pltpu.SemaphoreType.DMA((2,)),
    ],
)(_scatter_body)


def _mm_body(x_ref, a_ref, o_ref):
    acc = jnp.dot(x_ref[...], a_ref[...],
                  preferred_element_type=jnp.float32,
                  precision=lax.Precision.HIGHEST)
    o_ref[...] = jax.nn.sigmoid(acc)


BM = 256
BN = 512


def _matmul_tc(x, adj):
    m = x.shape[0]
    return pl.pallas_call(
        _mm_body,
        grid=(N_STATES // BN, m // BM),
        in_specs=[
            pl.BlockSpec((BM, N_STATES), lambda j, i: (i, 0)),
            pl.BlockSpec((N_STATES, BN), lambda j, i: (0, j)),
        ],
        out_specs=pl.BlockSpec((BM, BN), lambda j, i: (i, j)),
        out_shape=jax.ShapeDtypeStruct((m, N_STATES), jnp.float32),
    )(x, adj)


def kernel(x, rule_indices, rule_weights):
    body = rule_indices[0]
    head = rule_indices[1]
    adj = _scatter_sc(body, head, rule_weights)
    return _matmul_tc(x, adj)


# speedup vs baseline: 7.4773x; 1.2266x over previous
import functools

import jax
import jax.numpy as jnp
from jax import lax
from jax.experimental import pallas as pl
from jax.experimental.pallas import tpu as pltpu
from jax.experimental.pallas import tpu_sc as plsc

N_STATES = 4096
N_RULES = 1638400
ADJ_WORDS = N_STATES * N_STATES

NUM_SC = 2
NUM_TILES = 16
LANES = 16

W_PASS = 1507328
N_PASS = -(-ADJ_WORDS // W_PASS)

E_TILE = N_RULES // NUM_TILES
BE = 2048
N_PAIR = E_TILE // (2 * BE)
VPB = BE // LANES

ZBUF = 4096


def _scatter_body(body_hbm, head_hbm, w_hbm, adj_out, acc, bvec, hvec, wvec,
                  idx2d, w2d, zbuf, stage, ld_sem, st_sem):
    c = lax.axis_index("c")
    s = lax.axis_index("s")
    iota = lax.iota(jnp.int32, LANES)

    def _z(i, carry):
        zbuf[pl.ds(i * LANES, LANES)] = jnp.zeros((LANES,), jnp.float32)
        return carry
    lax.fori_loop(0, ZBUF // LANES, _z, 0)

    def _ld(j, p):
        eb = s * E_TILE + j * BE
        pltpu.async_copy(body_hbm.at[pl.ds(eb, BE)], bvec.at[p], ld_sem.at[p])
        pltpu.async_copy(head_hbm.at[pl.ds(eb, BE)], hvec.at[p], ld_sem.at[p])
        pltpu.async_copy(w_hbm.at[pl.ds(eb, BE)], wvec.at[p], ld_sem.at[p])

    def _ld_wait(j, p):
        eb = s * E_TILE + j * BE
        pltpu.make_async_copy(body_hbm.at[pl.ds(eb, BE)], bvec.at[p], ld_sem.at[p]).wait()
        pltpu.make_async_copy(head_hbm.at[pl.ds(eb, BE)], hvec.at[p], ld_sem.at[p]).wait()
        pltpu.make_async_copy(w_hbm.at[pl.ds(eb, BE)], wvec.at[p], ld_sem.at[p]).wait()

    def pass_body(k, carry):
        base = (k * NUM_SC + c) * W_PASS
        n_words = jnp.minimum(W_PASS, ADJ_WORDS - base)
        tile_share = n_words // NUM_TILES
        n_slabs = tile_share // ZBUF

        def _zero(j, carry2):
            off = pl.multiple_of(s * tile_share + j * ZBUF, ZBUF)
            pltpu.sync_copy(zbuf, acc.at[pl.ds(off, ZBUF)])
            return carry2
        lax.fori_loop(0, n_slabs, _zero, 0)
        plsc.subcore_barrier()

        def _compute(j, p):
            for v in range(VPB):
                off = v * LANES
                b16 = bvec[p, pl.ds(off, LANES)]
                h16 = hvec[p, pl.ds(off, LANES)]
                w16 = wvec[p, pl.ds(off, LANES)]
                local = (b16 << 12) + h16 - base
                inb = plsc.bitcast(local, jnp.uint32) < jnp.uint32(W_PASS)
                dummy = iota + off
                idx2d[p, v // 8, pl.ds((v % 8) * LANES, LANES)] = (
                    jnp.where(inb, local, dummy))
                w2d[p, v // 8, pl.ds((v % 8) * LANES, LANES)] = (
                    jnp.where(inb, w16, 0.0))

        def _scat_start(p):
            for g in range(BE // 128):
                pltpu.async_copy(w2d.at[p, g], acc.at[idx2d.at[p, g]],
                                 st_sem.at[p], add=True)

        def _scat_wait(p):
            for g in range(BE // 128):
                pltpu.make_async_copy(w2d.at[p, g], acc.at[idx2d.at[p, g]],
                                      st_sem.at[p]).wait()

        _ld(0, 0)

        def pair_body(i, carry2):
            _ld_wait(2 * i, 0)
            _ld(2 * i + 1, 1)

            @pl.when(i > 0)
            def _():
                _scat_wait(0)
            _compute(2 * i, 0)
            _scat_start(0)

            _ld_wait(2 * i + 1, 1)

            @pl.when(i < N_PAIR - 1)
            def _():
                _ld(2 * i + 2, 0)

            @pl.when(i > 0)
            def _():
                _scat_wait(1)
            _compute(2 * i + 1, 1)
            _scat_start(1)
            return carry2
        lax.fori_loop(0, N_PAIR, pair_body, 0)
        _scat_wait(0)
        _scat_wait(1)
        plsc.subcore_barrier()

        def _wout(j, carry2):
            off = pl.multiple_of(s * tile_share + j * ZBUF, ZBUF)
            row = (base + off) >> 12
            pltpu.sync_copy(acc.at[pl.ds(off, ZBUF)], stage)
            pltpu.sync_copy(stage, adj_out.at[row])
            return carry2
        lax.fori_loop(0, n_slabs, _wout, 0)
        plsc.subcore_barrier()
        return carry
    n_my_passes = (N_PASS + 1 - c) // 2
    lax.fori_loop(0, n_my_passes, pass_body, 0)


_scatter_sc = functools.partial(
    pl.kernel,
    out_type=jax.ShapeDtypeStruct((N_STATES, N_STATES), jnp.float32),
    mesh=plsc.VectorSubcoreMesh(core_axis_name="c", subcore_axis_name="s"),
    scratch_types=[
        pltpu.VMEM_SHARED((W_PASS,), jnp.float32),
        pltpu.VMEM((2, BE), jnp.int32),
        pltpu.VMEM((2, BE), jnp.int32),
        pltpu.VMEM((2, BE), jnp.float32),
        pltpu.VMEM((2, BE // 128, 128), jnp.int32),
        pltpu.VMEM((2, BE // 128, 128), jnp.float32),
        pltpu.VMEM((ZBUF,), jnp.float32),
        pltpu.VMEM((ZBUF,), jnp.float32),
        pltpu.SemaphoreType.DMA((2,)),
        pltpu.SemaphoreType.DMA((2,)),
    ],
)(_scatter_body)


def _mm_body(x_ref, a_ref, o_ref):
    a = a_ref[...]
    hi = a.astype(jnp.bfloat16)
    lo = (a - hi.astype(jnp.float32)).astype(jnp.bfloat16)
    xb = x_ref[...]
    acc = jnp.dot(xb, hi, preferred_element_type=jnp.float32)
    acc = acc + jnp.dot(xb, lo, preferred_element_type=jnp.float32)
    o_ref[...] = jax.nn.sigmoid(acc)


BM = 256
BN = 512


def _matmul_tc(x, adj):
    m = x.shape[0]
    return pl.pallas_call(
        _mm_body,
        grid=(N_STATES // BN, m // BM),
        in_specs=[
            pl.BlockSpec((BM, N_STATES), lambda j, i: (i, 0)),
            pl.BlockSpec((N_STATES, BN), lambda j, i: (0, j)),
        ],
        out_specs=pl.BlockSpec((BM, BN), lambda j, i: (i, j)),
        out_shape=jax.ShapeDtypeStruct((m, N_STATES), jnp.float32),
    )(x, adj)


def kernel(x, rule_indices, rule_weights):
    body = rule_indices[0]
    head = rule_indices[1]
    adj = _scatter_sc(body, head, rule_weights)
    return _matmul_tc(x.astype(jnp.bfloat16), adj)


# speedup vs baseline: 8.8284x; 1.1807x over previous
import functools

import jax
import jax.numpy as jnp
from jax import lax
from jax.experimental import pallas as pl
from jax.experimental.pallas import tpu as pltpu
from jax.experimental.pallas import tpu_sc as plsc

N_STATES = 4096
N_RULES = 1638400
ADJ_WORDS = N_STATES * N_STATES

NUM_SC = 2
NUM_TILES = 16
LANES = 16

W_PASS = 1507328
N_PASS = -(-ADJ_WORDS // W_PASS)

E_TILE = N_RULES // NUM_TILES
BE = 2048
N_PAIR = E_TILE // (2 * BE)
VPB = BE // LANES

ZBUF = 4096


def _scatter_body(body_hbm, head_hbm, w_hbm, adj_out, acc, bvec, hvec, wvec,
                  idx_a, idx_b, w_a, w_b, zbuf, stage, ld_sem, st_sem):
    c = lax.axis_index("c")
    s = lax.axis_index("s")
    iota = lax.iota(jnp.int32, LANES)

    def _z(i, carry):
        zbuf[pl.ds(i * LANES, LANES)] = jnp.zeros((LANES,), jnp.float32)
        return carry
    lax.fori_loop(0, ZBUF // LANES, _z, 0)

    def _ld(j, p):
        eb = s * E_TILE + j * BE
        pltpu.async_copy(body_hbm.at[pl.ds(eb, BE)], bvec.at[p], ld_sem.at[p])
        pltpu.async_copy(head_hbm.at[pl.ds(eb, BE)], hvec.at[p], ld_sem.at[p])
        pltpu.async_copy(w_hbm.at[pl.ds(eb, BE)], wvec.at[p], ld_sem.at[p])

    def _ld_wait(j, p):
        eb = s * E_TILE + j * BE
        pltpu.make_async_copy(body_hbm.at[pl.ds(eb, BE)], bvec.at[p], ld_sem.at[p]).wait()
        pltpu.make_async_copy(head_hbm.at[pl.ds(eb, BE)], hvec.at[p], ld_sem.at[p]).wait()
        pltpu.make_async_copy(w_hbm.at[pl.ds(eb, BE)], wvec.at[p], ld_sem.at[p]).wait()

    def pass_body(k, carry):
        base = (k * NUM_SC + c) * W_PASS
        n_words = jnp.minimum(W_PASS, ADJ_WORDS - base)
        tile_share = n_words // NUM_TILES
        n_slabs = tile_share // ZBUF

        def _zero(j, carry2):
            off = pl.multiple_of(s * tile_share + j * ZBUF, ZBUF)
            pltpu.sync_copy(zbuf, acc.at[pl.ds(off, ZBUF)])
            return carry2
        lax.fori_loop(0, n_slabs, _zero, 0)
        plsc.subcore_barrier()

        def _compute(j, p):
            for v in range(VPB):
                off = v * LANES
                b16 = bvec[p, pl.ds(off, LANES)]
                h16 = hvec[p, pl.ds(off, LANES)]
                w16 = wvec[p, pl.ds(off, LANES)]
                local = (b16 << 12) + h16 - base
                inb = plsc.bitcast(local, jnp.uint32) < jnp.uint32(W_PASS)
                dummy = iota + off
                idxbuf = idx_a if p == 0 else idx_b
                wbuf = w_a if p == 0 else w_b
                idxbuf[pl.ds(off, LANES)] = jnp.where(inb, local, dummy)
                wbuf[pl.ds(off, LANES)] = jnp.where(inb, w16, 0.0)

        def _scat_start(p):
            idxbuf = idx_a if p == 0 else idx_b
            wbuf = w_a if p == 0 else w_b
            pltpu.async_copy(wbuf, acc.at[idxbuf], st_sem.at[p], add=True)

        def _scat_wait(p):
            idxbuf = idx_a if p == 0 else idx_b
            wbuf = w_a if p == 0 else w_b
            pltpu.make_async_copy(wbuf, acc.at[idxbuf], st_sem.at[p]).wait()

        _ld(0, 0)

        def pair_body(i, carry2):
            _ld_wait(2 * i, 0)
            _ld(2 * i + 1, 1)

            @pl.when(i > 0)
            def _():
                _scat_wait(0)
            _compute(2 * i, 0)
            _scat_start(0)

            _ld_wait(2 * i + 1, 1)

            @pl.when(i < N_PAIR - 1)
            def _():
                _ld(2 * i + 2, 0)

            @pl.when(i > 0)
            def _():
                _scat_wait(1)
            _compute(2 * i + 1, 1)
            _scat_start(1)
            return carry2
        lax.fori_loop(0, N_PAIR, pair_body, 0)
        _scat_wait(0)
        _scat_wait(1)
        plsc.subcore_barrier()

        def _wout(j, carry2):
            off = pl.multiple_of(s * tile_share + j * ZBUF, ZBUF)
            row = (base + off) >> 12
            pltpu.sync_copy(acc.at[pl.ds(off, ZBUF)], stage)
            pltpu.sync_copy(stage, adj_out.at[row])
            return carry2
        lax.fori_loop(0, n_slabs, _wout, 0)
        plsc.subcore_barrier()
        return carry
    n_my_passes = (N_PASS + 1 - c) // 2
    lax.fori_loop(0, n_my_passes, pass_body, 0)


_scatter_sc = functools.partial(
    pl.kernel,
    out_type=jax.ShapeDtypeStruct((N_STATES, N_STATES), jnp.float32),
    mesh=plsc.VectorSubcoreMesh(core_axis_name="c", subcore_axis_name="s"),
    scratch_types=[
        pltpu.VMEM_SHARED((W_PASS,), jnp.float32),
        pltpu.VMEM((2, BE), jnp.int32),
        pltpu.VMEM((2, BE), jnp.int32),
        pltpu.VMEM((2, BE), jnp.float32),
        pltpu.VMEM((BE,), jnp.int32),
        pltpu.VMEM((BE,), jnp.int32),
        pltpu.VMEM((BE,), jnp.float32),
        pltpu.VMEM((BE,), jnp.float32),
        pltpu.VMEM((ZBUF,), jnp.float32),
        pltpu.VMEM((ZBUF,), jnp.float32),
        pltpu.SemaphoreType.DMA((2,)),
        pltpu.SemaphoreType.DMA((2,)),
    ],
)(_scatter_body)


def _mm_body(x_ref, a_ref, o_ref):
    a = a_ref[...]
    hi = a.astype(jnp.bfloat16)
    lo = (a - hi.astype(jnp.float32)).astype(jnp.bfloat16)
    xb = x_ref[...]
    acc = jnp.dot(xb, hi, preferred_element_type=jnp.float32)
    acc = acc + jnp.dot(xb, lo, preferred_element_type=jnp.float32)
    o_ref[...] = jax.nn.sigmoid(acc)


BM = 256
BN = 512


def _matmul_tc(x, adj):
    m = x.shape[0]
    return pl.pallas_call(
        _mm_body,
        grid=(N_STATES // BN, m // BM),
        in_specs=[
            pl.BlockSpec((BM, N_STATES), lambda j, i: (i, 0)),
            pl.BlockSpec((N_STATES, BN), lambda j, i: (0, j)),
        ],
        out_specs=pl.BlockSpec((BM, BN), lambda j, i: (i, j)),
        out_shape=jax.ShapeDtypeStruct((m, N_STATES), jnp.float32),
    )(x, adj)


def kernel(x, rule_indices, rule_weights):
    body = rule_indices[0]
    head = rule_indices[1]
    adj = _scatter_sc(body, head, rule_weights)
    return _matmul_tc(x.astype(jnp.bfloat16), adj)
